# Initial kernel scaffold; baseline (speedup 1.0000x reference)
import jax
import jax.numpy as jnp
from jax.experimental import pallas as pl


def kernel(x, regression, classification, anchors):
    raise NotImplementedError("write your pallas kernel here")


# speedup vs baseline: 5.7683x; 5.7683x over previous
import functools

import jax
import jax.numpy as jnp
from jax import lax
from jax.experimental import pallas as pl
from jax.experimental.pallas import tpu as pltpu

_THRESHOLD = 0.5
_NMS_THRESHOLD = 0.5
_MAX_DET = 100
_N = 20000
_R, _C = 8, 2500
_NEG_INF = float("-inf")


def _nms_kernel(an_ref, rg_ref, cls_ref,
                boxes_ref, scores_ref, classes_ref, ndet_ref):
    a0 = an_ref[0]
    a1 = an_ref[1]
    a2 = an_ref[2]
    a3 = an_ref[3]
    r0 = rg_ref[0]
    r1 = rg_ref[1]
    r2 = rg_ref[2]
    r3 = rg_ref[3]

    y_centers_a = (a0 + a2) / 2.0
    x_centers_a = (a1 + a3) / 2.0
    ha = a2 - a0
    wa = a3 - a1
    w = jnp.exp(r3) * wa
    h = jnp.exp(r2) * ha
    y_centers = r0 * ha + y_centers_a
    x_centers = r1 * wa + x_centers_a
    bx1 = jnp.maximum(x_centers - w / 2.0, 0.0)
    by1 = jnp.maximum(y_centers - h / 2.0, 0.0)
    bx2 = jnp.minimum(x_centers + w / 2.0, 511.0)
    by2 = jnp.minimum(y_centers + h / 2.0, 511.0)
    areas = jnp.maximum(bx2 - bx1, 0.0) * jnp.maximum(by2 - by1, 0.0)

    cls = cls_ref[...]
    max_scores = jnp.max(cls, axis=2)
    cit = lax.broadcasted_iota(jnp.int32, cls.shape, 2)
    class_idx = jnp.min(jnp.where(cls == max_scores[:, :, None], cit, 90),
                        axis=2)

    cur0 = jnp.where(max_scores > _THRESHOLD, max_scores, _NEG_INF)

    ri = lax.broadcasted_iota(jnp.int32, (_R, _C), 0)
    ci = lax.broadcasted_iota(jnp.int32, (_R, _C), 1)
    fidx = ri * _C + ci

    def body(t, carry):
        cur, cnt = carry
        m = jnp.max(cur)
        idx = jnp.min(jnp.where(cur == m, fidx, _N))
        oh = fidx == idx
        v = m > _NEG_INF

        bx1i = jnp.sum(jnp.where(oh, bx1, 0.0))
        by1i = jnp.sum(jnp.where(oh, by1, 0.0))
        bx2i = jnp.sum(jnp.where(oh, bx2, 0.0))
        by2i = jnp.sum(jnp.where(oh, by2, 0.0))
        cls_i = jnp.sum(jnp.where(oh, class_idx, 0))
        areas_i = jnp.maximum(bx2i - bx1i, 0.0) * jnp.maximum(by2i - by1i, 0.0)

        xx1 = jnp.maximum(bx1, bx1i)
        yy1 = jnp.maximum(by1, by1i)
        xx2 = jnp.minimum(bx2, bx2i)
        yy2 = jnp.minimum(by2, by2i)
        inter = jnp.maximum(xx2 - xx1, 0.0) * jnp.maximum(yy2 - yy1, 0.0)
        iou = inter / (areas + areas_i - inter + 1e-8)
        supp = (iou > _NMS_THRESHOLD) | oh
        cur = jnp.where(jnp.logical_and(v, supp), _NEG_INF, cur)

        boxes_ref[t, 0] = jnp.where(v, bx1i, 0.0)
        boxes_ref[t, 1] = jnp.where(v, by1i, 0.0)
        boxes_ref[t, 2] = jnp.where(v, bx2i, 0.0)
        boxes_ref[t, 3] = jnp.where(v, by2i, 0.0)
        scores_ref[t] = jnp.where(v, m, 0.0)
        classes_ref[t] = jnp.where(v, cls_i, 0)
        return cur, cnt + v.astype(jnp.int32)

    _, cnt = lax.fori_loop(0, _MAX_DET, body, (cur0, jnp.int32(0)))
    ndet_ref[0] = cnt


@jax.jit
def _run(anchors_t, regression_t, cls3):
    return pl.pallas_call(
        _nms_kernel,
        out_shape=(
            jax.ShapeDtypeStruct((_MAX_DET, 4), jnp.float32),
            jax.ShapeDtypeStruct((_MAX_DET,), jnp.float32),
            jax.ShapeDtypeStruct((_MAX_DET,), jnp.int32),
            jax.ShapeDtypeStruct((1,), jnp.int32),
        ),
        in_specs=[
            pl.BlockSpec(memory_space=pltpu.VMEM),
            pl.BlockSpec(memory_space=pltpu.VMEM),
            pl.BlockSpec(memory_space=pltpu.VMEM),
        ],
        out_specs=(
            pl.BlockSpec(memory_space=pltpu.SMEM),
            pl.BlockSpec(memory_space=pltpu.SMEM),
            pl.BlockSpec(memory_space=pltpu.SMEM),
            pl.BlockSpec(memory_space=pltpu.SMEM),
        ),
    )(anchors_t, regression_t, cls3)


def kernel(x, regression, classification, anchors):
    del x
    anchors_t = anchors[0].T.reshape(4, _R, _C)
    regression_t = regression[0].T.reshape(4, _R, _C)
    cls3 = classification.reshape(_R, _C, 90)
    boxes, scores, classes, ndet = _run(anchors_t, regression_t, cls3)
    return boxes, scores, classes, ndet[0]


# speedup vs baseline: 7.7614x; 1.3455x over previous
import functools

import jax
import jax.numpy as jnp
from jax import lax
from jax.experimental import pallas as pl
from jax.experimental.pallas import tpu as pltpu
from jax.experimental.pallas import tpu_sc as plsc

_THRESHOLD = 0.5
_NMS_THRESHOLD = 0.5
_MAX_DET = 100
_N = 20000
_NSUB = 16
_P = 1280
_NPAD = _NSUB * _P
_CH = _P // 16
_OUTP = 112
_NEG = float("-inf")


def _prep_kernel(an_ref, rg_ref, cls_ref,
                 cur_ref, bx1_ref, by1_ref, bx2_ref, by2_ref, ar_ref, ci_ref):
    a0 = an_ref[0]
    a1 = an_ref[1]
    a2 = an_ref[2]
    a3 = an_ref[3]
    r0 = rg_ref[0]
    r1 = rg_ref[1]
    r2 = rg_ref[2]
    r3 = rg_ref[3]

    y_centers_a = (a0 + a2) / 2.0
    x_centers_a = (a1 + a3) / 2.0
    ha = a2 - a0
    wa = a3 - a1
    w = jnp.exp(r3) * wa
    h = jnp.exp(r2) * ha
    y_centers = r0 * ha + y_centers_a
    x_centers = r1 * wa + x_centers_a
    bx1 = jnp.maximum(x_centers - w / 2.0, 0.0)
    by1 = jnp.maximum(y_centers - h / 2.0, 0.0)
    bx2 = jnp.minimum(x_centers + w / 2.0, 511.0)
    by2 = jnp.minimum(y_centers + h / 2.0, 511.0)

    bx1_ref[...] = bx1
    by1_ref[...] = by1
    bx2_ref[...] = bx2
    by2_ref[...] = by2
    ar_ref[...] = jnp.maximum(bx2 - bx1, 0.0) * jnp.maximum(by2 - by1, 0.0)

    cls = cls_ref[...]
    max_scores = jnp.max(cls, axis=2)
    cit = lax.broadcasted_iota(jnp.int32, cls.shape, 2)
    ci_ref[...] = jnp.min(jnp.where(cls == max_scores[:, :, None], cit, 90),
                          axis=2)
    cur_ref[...] = jnp.where(max_scores > _THRESHOLD, max_scores, _NEG)


def _prep(anchors_t, regression_t, cls3):
    shp = jax.ShapeDtypeStruct((_NSUB, _P), jnp.float32)
    shpi = jax.ShapeDtypeStruct((_NSUB, _P), jnp.int32)
    return pl.pallas_call(
        _prep_kernel,
        out_shape=(shp, shp, shp, shp, shp, shp, shpi),
        in_specs=[pl.BlockSpec(memory_space=pltpu.VMEM)] * 3,
    )(anchors_t, regression_t, cls3)


def _sc_nms_body(cur_hbm, bx1_hbm, by1_hbm, bx2_hbm, by2_hbm, ar_hbm, ci_hbm,
                 xs_hbm, ys_hbm, x2s_hbm, y2s_hbm, ss_hbm, cs_hbm, nd_hbm,
                 cur_v, x1_v, y1_v, x2_v, y2_v, ar_v, cl_v,
                 stage_v, all_v, xs_v, ys_v, x2s_v, y2s_v, ss_v, cs_v, nd_v,
                 cand_tbl):
    s = lax.axis_index("s")
    lane = lax.iota(jnp.int32, 16)
    neg = jnp.float32(_NEG)

    pltpu.sync_copy(cur_hbm.at[s], cur_v)
    pltpu.sync_copy(bx1_hbm.at[s], x1_v)
    pltpu.sync_copy(by1_hbm.at[s], y1_v)
    pltpu.sync_copy(bx2_hbm.at[s], x2_v)
    pltpu.sync_copy(by2_hbm.at[s], y2_v)
    pltpu.sync_copy(ar_hbm.at[s], ar_v)
    pltpu.sync_copy(ci_hbm.at[s], cl_v)

    def lane_bcast(vec, j):
        return jnp.full((16,), jnp.max(jnp.where(lane == j, vec, neg)),
                        jnp.float32)

    def pick_body(t, cnt):
        def am_body(i, carry):
            vm, bc = carry
            v = cur_v[pl.ds(i * 16, 16)]
            better = v > vm
            ib = jnp.full((16,), i, jnp.int32)
            return jnp.where(better, v, vm), jnp.where(better, ib, bc)

        vm, bc = lax.fori_loop(
            0, _CH, am_body,
            (jnp.full((16,), neg, jnp.float32), jnp.zeros((16,), jnp.int32)))
        m = jnp.max(vm)
        li = jnp.min(jnp.where(vm == m, bc * 16 + lane, _P))
        gidx = s * _P + li

        loff = (li // 16) * 16
        llane = li - loff
        gx1 = lane_bcast(x1_v[pl.ds(loff, 16)], llane)
        gy1 = lane_bcast(y1_v[pl.ds(loff, 16)], llane)
        gx2 = lane_bcast(x2_v[pl.ds(loff, 16)], llane)
        gy2 = lane_bcast(y2_v[pl.ds(loff, 16)], llane)
        gar = lane_bcast(ar_v[pl.ds(loff, 16)], llane)
        gcl = lane_bcast(cl_v[pl.ds(loff, 16)].astype(jnp.float32), llane)
        mb = jnp.full((16,), m, jnp.float32)
        gf = jnp.full((16,), gidx, jnp.int32).astype(jnp.float32)
        cand = jnp.where(lane == 0, mb,
               jnp.where(lane == 1, gf,
               jnp.where(lane == 2, gx1,
               jnp.where(lane == 3, gy1,
               jnp.where(lane == 4, gx2,
               jnp.where(lane == 5, gy2,
               jnp.where(lane == 6, gar, gcl)))))))
        stage_v[...] = cand
        pltpu.sync_copy(stage_v, cand_tbl.at[pl.ds(s * 16, 16)])
        plsc.subcore_barrier()
        pltpu.sync_copy(cand_tbl, all_v)
        plsc.subcore_barrier()

        def win_body(j, carry):
            bs, bg, brow = carry
            row = all_v[pl.ds(j * 16, 16)]
            sj = jnp.max(jnp.where(lane == 0, row, neg))
            gj = jnp.max(jnp.where(lane == 1, row, neg))
            better = (sj > bs) | ((sj == bs) & (gj < bg))
            return (jnp.where(better, sj, bs), jnp.where(better, gj, bg),
                    jnp.where(better, row, brow))

        wm, gw, brow = lax.fori_loop(
            0, _NSUB, win_body,
            (neg, jnp.float32(3e9), jnp.zeros((16,), jnp.float32)))
        wx1 = lane_bcast(brow, 2)
        wy1 = lane_bcast(brow, 3)
        wx2 = lane_bcast(brow, 4)
        wy2 = lane_bcast(brow, 5)
        war = lane_bcast(brow, 6)
        wcl = lane_bcast(brow, 7)
        validv = jnp.full((16,), wm, jnp.float32) > neg

        base = s * _P

        def sup_body(i, carry):
            x1c = x1_v[pl.ds(i * 16, 16)]
            y1c = y1_v[pl.ds(i * 16, 16)]
            x2c = x2_v[pl.ds(i * 16, 16)]
            y2c = y2_v[pl.ds(i * 16, 16)]
            arc = ar_v[pl.ds(i * 16, 16)]
            cc = cur_v[pl.ds(i * 16, 16)]
            xx1 = jnp.maximum(x1c, wx1)
            yy1 = jnp.maximum(y1c, wy1)
            xx2 = jnp.minimum(x2c, wx2)
            yy2 = jnp.minimum(y2c, wy2)
            inter = jnp.maximum(xx2 - xx1, 0.0) * jnp.maximum(yy2 - yy1, 0.0)
            iou = inter / (arc + war - inter + 1e-8)
            gidc = (jnp.full((16,), base + i * 16, jnp.int32) + lane
                    ).astype(jnp.float32)
            supp = (iou > _NMS_THRESHOLD) | (gidc == gw)
            cur_v[pl.ds(i * 16, 16)] = jnp.where(validv & supp, neg, cc)
            return carry

        lax.fori_loop(0, _CH, sup_body, 0)

        @pl.when(s == 0)
        def _():
            off = (t // 16) * 16
            lsel = lane == (t - off)

            def put(ref, valvec):
                old = ref[pl.ds(off, 16)]
                ref[pl.ds(off, 16)] = jnp.where(
                    lsel, jnp.where(validv, valvec, 0.0), old)

            put(xs_v, wx1)
            put(ys_v, wy1)
            put(x2s_v, wx2)
            put(y2s_v, wy2)
            put(ss_v, jnp.full((16,), wm, jnp.float32))
            oldc = cs_v[pl.ds(off, 16)]
            cs_v[pl.ds(off, 16)] = jnp.where(
                lsel, jnp.where(validv, wcl.astype(jnp.int32), 0), oldc)

        return cnt + jnp.where(wm > neg, 1, 0).astype(jnp.int32)

    cnt = lax.fori_loop(0, _MAX_DET, pick_body, jnp.int32(0))

    @pl.when(s == 0)
    def _():
        nd_v[...] = jnp.where(lane == 0, jnp.full((16,), cnt, jnp.int32), 0)
        pltpu.sync_copy(xs_v, xs_hbm)
        pltpu.sync_copy(ys_v, ys_hbm)
        pltpu.sync_copy(x2s_v, x2s_hbm)
        pltpu.sync_copy(y2s_v, y2s_hbm)
        pltpu.sync_copy(ss_v, ss_hbm)
        pltpu.sync_copy(cs_v, cs_hbm)
        pltpu.sync_copy(nd_v, nd_hbm)


def _sc_nms(cur, bx1, by1, bx2, by2, ar, ci, *, interpret=False):
    f32 = jnp.float32
    outs = (
        jax.ShapeDtypeStruct((_OUTP,), f32),
        jax.ShapeDtypeStruct((_OUTP,), f32),
        jax.ShapeDtypeStruct((_OUTP,), f32),
        jax.ShapeDtypeStruct((_OUTP,), f32),
        jax.ShapeDtypeStruct((_OUTP,), f32),
        jax.ShapeDtypeStruct((_OUTP,), jnp.int32),
        jax.ShapeDtypeStruct((16,), jnp.int32),
    )
    scratch = [
        pltpu.VMEM((_P,), f32),
        pltpu.VMEM((_P,), f32),
        pltpu.VMEM((_P,), f32),
        pltpu.VMEM((_P,), f32),
        pltpu.VMEM((_P,), f32),
        pltpu.VMEM((_P,), f32),
        pltpu.VMEM((_P,), jnp.int32),
        pltpu.VMEM((16,), f32),
        pltpu.VMEM((16 * _NSUB,), f32),
        pltpu.VMEM((_OUTP,), f32),
        pltpu.VMEM((_OUTP,), f32),
        pltpu.VMEM((_OUTP,), f32),
        pltpu.VMEM((_OUTP,), f32),
        pltpu.VMEM((_OUTP,), f32),
        pltpu.VMEM((_OUTP,), jnp.int32),
        pltpu.VMEM((16,), jnp.int32),
        pltpu.VMEM_SHARED((16 * _NSUB,), f32),
    ]
    mesh = plsc.VectorSubcoreMesh(core_axis_name="c", subcore_axis_name="s",
                                  num_cores=1, num_subcores=_NSUB)
    call = pl.kernel(_sc_nms_body, out_type=outs, mesh=mesh,
                     scratch_types=scratch, interpret=interpret,
                     compiler_params=pltpu.CompilerParams(
                         needs_layout_passes=False))
    return call(cur, bx1, by1, bx2, by2, ar, ci)


def _nms_full(x, regression, classification, anchors, *, interpret=False):
    del x
    pad = _NPAD - _N
    anchors_t = jnp.pad(anchors[0], ((0, pad), (0, 0))).T.reshape(4, _NSUB, _P)
    regression_t = jnp.pad(regression[0], ((0, pad), (0, 0))).T.reshape(
        4, _NSUB, _P)
    cls3 = jnp.pad(classification[0], ((0, pad), (0, 0))).reshape(
        _NSUB, _P, 90)
    cur, bx1, by1, bx2, by2, ar, ci = _prep(anchors_t, regression_t, cls3)
    xs, ys, x2s, y2s, ss, cs, nd = _sc_nms(cur, bx1, by1, bx2, by2, ar, ci,
                                           interpret=interpret)
    boxes = jnp.stack([xs[:_MAX_DET], ys[:_MAX_DET],
                       x2s[:_MAX_DET], y2s[:_MAX_DET]], axis=-1)
    return boxes, ss[:_MAX_DET], cs[:_MAX_DET], nd[0]


@jax.jit
def _run(x, regression, classification, anchors):
    return _nms_full(x, regression, classification, anchors)


def kernel(x, regression, classification, anchors):
    return _run(x, regression, classification, anchors)
